# Initial kernel scaffold; baseline (speedup 1.0000x reference)
import jax
import jax.numpy as jnp
from jax.experimental import pallas as pl


def kernel(edge_type, table):
    raise NotImplementedError("write your pallas kernel here")


# speedup vs baseline: 1.1313x; 1.1313x over previous
import functools

import jax
import jax.numpy as jnp
from jax import lax
from jax.experimental import pallas as pl
from jax.experimental.pallas import tpu as pltpu
from jax.experimental.pallas import tpu_sc as plsc

E = 800000
D = 64
NUM_CORES = 2
NUM_SUBCORES = 16
NW = NUM_CORES * NUM_SUBCORES
CP = 128
T = (E // 2) // CP
Q, R = divmod(T, NW)
MAXT = Q + 1
GROUPS = MAXT * CP // 16


@jax.jit
def _sc_embed(idx, ptab):
    mesh = plsc.VectorSubcoreMesh(core_axis_name="c", subcore_axis_name="s")

    @functools.partial(
        pl.kernel,
        mesh=mesh,
        out_type=jax.ShapeDtypeStruct((E // 2, 2 * D), jnp.float32),
        scratch_types=[
            pltpu.VMEM((MAXT * 2 * CP,), jnp.int32),
            pltpu.VMEM((MAXT * CP,), jnp.int32),
            pltpu.VMEM((CP, 2 * D), jnp.float32),
            pltpu.SemaphoreType.DMA,
        ],
        compiler_params=pltpu.CompilerParams(needs_layout_passes=False),
    )
    def k(idx_hbm, ptab_hbm, out_hbm, idx_v, pair_v, rows_v, sem):
        wid = lax.axis_index("s") * NUM_CORES + lax.axis_index("c")
        t0 = wid * Q + jnp.minimum(wid, R)
        nt = Q + jnp.where(wid < R, 1, 0)
        start = jnp.minimum(t0, T - MAXT)
        shift = t0 - start

        pltpu.sync_copy(idx_hbm.at[pl.ds(start * 2 * CP, MAXT * 2 * CP)], idx_v)

        two_iota = lax.iota(jnp.int32, 16) * 2

        def pair_body(g, carry):
            pos = two_iota + g * 32
            ev = plsc.load_gather(idx_v, [pos])
            od = plsc.load_gather(idx_v, [pos + 1])
            pair_v[pl.ds(g * 16, 16)] = jnp.bitwise_and(ev * 4 + od, 15)
            return carry

        lax.fori_loop(0, GROUPS, pair_body, 0)

        def xfer_body(ci, carry):
            li = shift + ci
            pltpu.async_copy(
                ptab_hbm.at[pair_v.at[pl.ds(li * CP, CP)]], rows_v, sem
            ).wait()
            pltpu.sync_copy(rows_v, out_hbm.at[pl.ds((t0 + ci) * CP, CP)])
            return carry

        lax.fori_loop(0, nt, xfer_body, 0)

    return k(idx, ptab)


def kernel(edge_type, table):
    idx = edge_type.astype(jnp.int32)
    ptab = jnp.concatenate(
        [jnp.repeat(table, 4, axis=0), jnp.tile(table, (4, 1))], axis=1
    )
    out2 = _sc_embed(idx, ptab)
    return out2.reshape(E, D)
